# Initial kernel scaffold; baseline (speedup 1.0000x reference)
import jax
import jax.numpy as jnp
from jax.experimental import pallas as pl


def kernel(indices, weight):
    raise NotImplementedError("write your pallas kernel here")


# speedup vs baseline: 2.7688x; 2.7688x over previous
import functools

import jax
import jax.numpy as jnp
from jax import lax
from jax.experimental import pallas as pl
from jax.experimental.pallas import tpu as pltpu
from jax.experimental.pallas import tpu_sc as plsc

VOCAB = 100000
D = 128
B_TOTAL = 4096 * 50

_info = plsc.get_sparse_core_info()
NC = _info.num_cores
NS = _info.num_subcores
NW = NC * NS
B_PER_W = B_TOTAL // NW
CHUNK = 128
N_CHUNKS = B_PER_W // CHUNK


def _gather_body(idx_hbm, table_hbm, out_hbm, idx_v, rows_v, sem):
    wid = lax.axis_index("s") * NC + lax.axis_index("c")
    base = wid * B_PER_W

    def chunk(i, carry):
        off = base + i * CHUNK
        pltpu.sync_copy(idx_hbm.at[pl.ds(off, CHUNK)], idx_v)
        pltpu.async_copy(table_hbm.at[idx_v], rows_v, sem).wait()
        pltpu.sync_copy(rows_v, out_hbm.at[pl.ds(off, CHUNK)])
        return carry

    lax.fori_loop(0, N_CHUNKS, chunk, 0)


@jax.jit
def _embed(indices_flat, weight):
    mesh = plsc.VectorSubcoreMesh(core_axis_name="c", subcore_axis_name="s")
    f = functools.partial(
        pl.kernel,
        out_type=jax.ShapeDtypeStruct((B_TOTAL, D), jnp.float32),
        mesh=mesh,
        scratch_types=[
            pltpu.VMEM((CHUNK,), jnp.int32),
            pltpu.VMEM((CHUNK, D), jnp.float32),
            pltpu.SemaphoreType.DMA,
        ],
    )(_gather_body)
    return f(indices_flat, weight)


def kernel(indices, weight):
    out = _embed(indices.reshape(-1), weight)
    return out.reshape(indices.shape + (D,))


# speedup vs baseline: 3.3571x; 1.2125x over previous
import functools

import jax
import jax.numpy as jnp
from jax import lax
from jax.experimental import pallas as pl
from jax.experimental.pallas import tpu as pltpu
from jax.experimental.pallas import tpu_sc as plsc

VOCAB = 100000
D = 128
B_TOTAL = 4096 * 50

_info = plsc.get_sparse_core_info()
NC = _info.num_cores
NS = _info.num_subcores
NW = NC * NS
B_PER_W = B_TOTAL // NW
CHUNK = 128
N_CHUNKS = B_PER_W // CHUNK
NBUF = 5
N_ROUNDS = N_CHUNKS // NBUF


def _gather_body(idx_hbm, table_hbm, out_hbm, idx_v, rows_v, gsems, wsems):
    wid = lax.axis_index("s") * NC + lax.axis_index("c")
    base = wid * B_PER_W

    pltpu.sync_copy(idx_hbm.at[wid], idx_v)

    for b in range(NBUF):
        pltpu.async_copy(table_hbm.at[idx_v.at[b]], rows_v.at[b], gsems.at[b])

    def round_(r, carry):
        for b in range(NBUF):
            g = r * NBUF + b
            out_slice = out_hbm.at[pl.ds(base + g * CHUNK, CHUNK)]
            pltpu.make_async_copy(table_hbm.at[idx_v.at[g]], rows_v.at[b],
                                  gsems.at[b]).wait()
            pltpu.async_copy(rows_v.at[b], out_slice, wsems.at[b])
            pltpu.make_async_copy(rows_v.at[b], out_slice, wsems.at[b]).wait()

            @pl.when(r < N_ROUNDS - 1)
            def _():
                pltpu.async_copy(table_hbm.at[idx_v.at[g + NBUF]],
                                 rows_v.at[b], gsems.at[b])
        return carry

    lax.fori_loop(0, N_ROUNDS, round_, 0)


@jax.jit
def _embed(indices_w, weight):
    mesh = plsc.VectorSubcoreMesh(core_axis_name="c", subcore_axis_name="s")
    f = functools.partial(
        pl.kernel,
        out_type=jax.ShapeDtypeStruct((B_TOTAL, D), jnp.float32),
        mesh=mesh,
        scratch_types=[
            pltpu.VMEM((N_CHUNKS, CHUNK), jnp.int32),
            pltpu.VMEM((NBUF, CHUNK, D), jnp.float32),
            pltpu.SemaphoreType.DMA((NBUF,)),
            pltpu.SemaphoreType.DMA((NBUF,)),
        ],
    )(_gather_body)
    return f(indices_w, weight)


def kernel(indices, weight):
    out = _embed(indices.reshape(NW, N_CHUNKS, CHUNK), weight)
    return out.reshape(indices.shape + (D,))


# speedup vs baseline: 5.9967x; 1.7863x over previous
import functools

import jax
import jax.numpy as jnp
from jax import lax
from jax.experimental import pallas as pl
from jax.experimental.pallas import tpu as pltpu
from jax.experimental.pallas import tpu_sc as plsc

VOCAB = 100000
D = 128
BATCH = 4096
SEQ = 50
SEQ_PAD = 64

_info = plsc.get_sparse_core_info()
NC = _info.num_cores
NS = _info.num_subcores
NW = NC * NS
ROWS_PER_W = BATCH // NW
NBUF = 8
N_ROUNDS = ROWS_PER_W // NBUF


def _gather_body(idx_hbm, table_hbm, out_hbm, idx_v, rows_v, gsems, wsems):
    wid = lax.axis_index("s") * NC + lax.axis_index("c")
    row0 = wid * ROWS_PER_W

    pltpu.sync_copy(idx_hbm.at[pl.ds(row0 * SEQ_PAD, ROWS_PER_W * SEQ_PAD)],
                    idx_v)

    def idx_slice(r):
        off = pl.multiple_of(r * SEQ_PAD, SEQ_PAD)
        return idx_v.at[pl.ds(off, SEQ)]

    for b in range(NBUF):
        pltpu.async_copy(table_hbm.at[idx_slice(b)], rows_v.at[b], gsems.at[b])

    def round_(rnd, carry):
        for b in range(NBUF):
            r = rnd * NBUF + b
            out_slice = out_hbm.at[row0 + r]
            pltpu.make_async_copy(table_hbm.at[idx_slice(r)], rows_v.at[b],
                                  gsems.at[b]).wait()
            pltpu.async_copy(rows_v.at[b], out_slice, wsems.at[b])
            pltpu.make_async_copy(rows_v.at[b], out_slice, wsems.at[b]).wait()

            @pl.when(rnd < N_ROUNDS - 1)
            def _():
                pltpu.async_copy(table_hbm.at[idx_slice(r + NBUF)],
                                 rows_v.at[b], gsems.at[b])
        return carry

    lax.fori_loop(0, N_ROUNDS, round_, 0)


@jax.jit
def _embed(indices_pad_flat, weight):
    mesh = plsc.VectorSubcoreMesh(core_axis_name="c", subcore_axis_name="s")
    f = functools.partial(
        pl.kernel,
        out_type=jax.ShapeDtypeStruct((BATCH, SEQ, D), jnp.float32),
        mesh=mesh,
        scratch_types=[
            pltpu.VMEM((ROWS_PER_W * SEQ_PAD,), jnp.int32),
            pltpu.VMEM((NBUF, SEQ, D), jnp.float32),
            pltpu.SemaphoreType.DMA((NBUF,)),
            pltpu.SemaphoreType.DMA((NBUF,)),
        ],
        compiler_params=pltpu.CompilerParams(use_tc_tiling_on_sc=True),
    )(_gather_body)
    return f(indices_pad_flat, weight)


def kernel(indices, weight):
    idx_pad = jnp.pad(indices, ((0, 0), (0, SEQ_PAD - SEQ))).reshape(-1)
    return _embed(idx_pad, weight)


# speedup vs baseline: 10.5149x; 1.7535x over previous
import functools

import jax
import jax.numpy as jnp
from jax import lax
from jax.experimental import pallas as pl
from jax.experimental.pallas import tpu as pltpu
from jax.experimental.pallas import tpu_sc as plsc

VOCAB = 100000
D = 128
BATCH = 4096
SEQ = 50
B_TOTAL = BATCH * SEQ

_info = plsc.get_sparse_core_info()
NC = _info.num_cores
NS = _info.num_subcores
NW = NC * NS
B_PER_W = B_TOTAL // NW
CHUNK = 128
N_CHUNKS = B_PER_W // CHUNK
NBUF = 5
N_ROUNDS = N_CHUNKS // NBUF


def _gather_body(idx_hbm, table_hbm, out_hbm, idx_v, rows_v, gsems, wsems):
    wid = lax.axis_index("s") * NC + lax.axis_index("c")
    base = wid * B_PER_W

    pltpu.sync_copy(idx_hbm.at[wid], idx_v)

    for b in range(NBUF):
        pltpu.async_copy(table_hbm.at[idx_v.at[b]], rows_v.at[b], gsems.at[b])

    def round_(rnd, carry):
        for b in range(NBUF):
            g = rnd * NBUF + b
            out_slice = out_hbm.at[pl.ds(base + g * CHUNK, CHUNK)]
            pltpu.make_async_copy(table_hbm.at[idx_v.at[g]], rows_v.at[b],
                                  gsems.at[b]).wait()
            pltpu.async_copy(rows_v.at[b], out_slice, wsems.at[b])
            pltpu.make_async_copy(rows_v.at[b], out_slice, wsems.at[b]).wait()

            @pl.when(rnd < N_ROUNDS - 1)
            def _():
                pltpu.async_copy(table_hbm.at[idx_v.at[g + NBUF]],
                                 rows_v.at[b], gsems.at[b])
        return carry

    lax.fori_loop(0, N_ROUNDS, round_, 0)


@jax.jit
def _embed(indices_tmajor, weight):
    mesh = plsc.VectorSubcoreMesh(core_axis_name="c", subcore_axis_name="s")
    f = functools.partial(
        pl.kernel,
        out_type=jax.ShapeDtypeStruct((B_TOTAL, D), jnp.float32),
        mesh=mesh,
        scratch_types=[
            pltpu.VMEM((N_CHUNKS, CHUNK), jnp.int32),
            pltpu.VMEM((NBUF, CHUNK, D), jnp.float32),
            pltpu.SemaphoreType.DMA((NBUF,)),
            pltpu.SemaphoreType.DMA((NBUF,)),
        ],
    )(_gather_body)
    out2d = f(indices_tmajor, weight)
    return out2d.reshape(SEQ, BATCH, D).transpose(1, 0, 2)


def kernel(indices, weight):
    idx_tmajor = indices.T.reshape(NW, N_CHUNKS, CHUNK)
    return _embed(idx_tmajor, weight)


# speedup vs baseline: 10.7763x; 1.0249x over previous
import functools

import jax
import jax.numpy as jnp
from jax import lax
from jax.experimental import pallas as pl
from jax.experimental.pallas import tpu as pltpu
from jax.experimental.pallas import tpu_sc as plsc

VOCAB = 100000
D = 128
BATCH = 4096
SEQ = 50
B_TOTAL = BATCH * SEQ

_info = plsc.get_sparse_core_info()
NC = _info.num_cores
NS = _info.num_subcores
NW = NC * NS
B_PER_W = B_TOTAL // NW
CHUNK = 128
N_CHUNKS = B_PER_W // CHUNK
NBUF = 5
N_ROUNDS = N_CHUNKS // NBUF


def _gather_body(idx_hbm, table_hbm, out_hbm, idx_v, rows_v, gsems, wsems):
    wid = lax.axis_index("s") * NC + lax.axis_index("c")

    pltpu.sync_copy(idx_hbm.at[:, pl.ds(wid * CHUNK, CHUNK)], idx_v)

    for b in range(NBUF):
        pltpu.async_copy(table_hbm.at[idx_v.at[b]], rows_v.at[b], gsems.at[b])

    def round_(rnd, carry):
        for b in range(NBUF):
            g = rnd * NBUF + b
            out_slice = out_hbm.at[pl.ds(g * BATCH + wid * CHUNK, CHUNK)]
            pltpu.make_async_copy(table_hbm.at[idx_v.at[g]], rows_v.at[b],
                                  gsems.at[b]).wait()
            pltpu.async_copy(rows_v.at[b], out_slice, wsems.at[b])
            pltpu.make_async_copy(rows_v.at[b], out_slice, wsems.at[b]).wait()

            @pl.when(rnd < N_ROUNDS - 1)
            def _():
                pltpu.async_copy(table_hbm.at[idx_v.at[g + NBUF]],
                                 rows_v.at[b], gsems.at[b])
        return carry

    lax.fori_loop(0, N_ROUNDS, round_, 0)


@jax.jit
def _embed(indices_tmajor, weight):
    mesh = plsc.VectorSubcoreMesh(core_axis_name="c", subcore_axis_name="s")
    f = functools.partial(
        pl.kernel,
        out_type=jax.ShapeDtypeStruct((B_TOTAL, D), jnp.float32),
        mesh=mesh,
        scratch_types=[
            pltpu.VMEM((SEQ, CHUNK), jnp.int32),
            pltpu.VMEM((NBUF, CHUNK, D), jnp.float32),
            pltpu.SemaphoreType.DMA((NBUF,)),
            pltpu.SemaphoreType.DMA((NBUF,)),
        ],
    )(_gather_body)
    out2d = f(indices_tmajor, weight)
    return out2d.reshape(SEQ, BATCH, D).transpose(1, 0, 2)


def kernel(indices, weight):
    return _embed(indices.T, weight)


# speedup vs baseline: 10.8343x; 1.0054x over previous
import functools

import jax
import jax.numpy as jnp
from jax import lax
from jax.experimental import pallas as pl
from jax.experimental.pallas import tpu as pltpu
from jax.experimental.pallas import tpu_sc as plsc

VOCAB = 100000
D = 128
BATCH = 4096
SEQ = 50
B_TOTAL = BATCH * SEQ

_info = plsc.get_sparse_core_info()
NC = _info.num_cores
NS = _info.num_subcores
NW = NC * NS
B_PER_W = B_TOTAL // NW
CHUNK = 128
N_CHUNKS = B_PER_W // CHUNK
NBUF = 7
N_ROUNDS = -(-N_CHUNKS // NBUF)


def _gather_body(idx_hbm, table_hbm, out_hbm, idx_v, rows_v, gsems, wsems):
    wid = lax.axis_index("s") * NC + lax.axis_index("c")

    pltpu.sync_copy(idx_hbm.at[:, pl.ds(wid * CHUNK, CHUNK)], idx_v)

    for b in range(NBUF):
        pltpu.async_copy(table_hbm.at[idx_v.at[b]], rows_v.at[b], gsems.at[b])

    def round_(rnd, carry):
        for b in range(NBUF):
            g = rnd * NBUF + b

            @pl.when(g < N_CHUNKS)
            def _():
                out_slice = out_hbm.at[pl.ds(g * BATCH + wid * CHUNK, CHUNK)]
                pltpu.make_async_copy(table_hbm.at[idx_v.at[g]], rows_v.at[b],
                                      gsems.at[b]).wait()
                pltpu.async_copy(rows_v.at[b], out_slice, wsems.at[b])
                pltpu.make_async_copy(rows_v.at[b], out_slice,
                                      wsems.at[b]).wait()

                @pl.when(g + NBUF < N_CHUNKS)
                def _():
                    pltpu.async_copy(table_hbm.at[idx_v.at[g + NBUF]],
                                     rows_v.at[b], gsems.at[b])
        return carry

    lax.fori_loop(0, N_ROUNDS, round_, 0)


@jax.jit
def _embed(indices_tmajor, weight):
    mesh = plsc.VectorSubcoreMesh(core_axis_name="c", subcore_axis_name="s")
    f = functools.partial(
        pl.kernel,
        out_type=jax.ShapeDtypeStruct((B_TOTAL, D), jnp.float32),
        mesh=mesh,
        scratch_types=[
            pltpu.VMEM((SEQ, CHUNK), jnp.int32),
            pltpu.VMEM((NBUF, CHUNK, D), jnp.float32),
            pltpu.SemaphoreType.DMA((NBUF,)),
            pltpu.SemaphoreType.DMA((NBUF,)),
        ],
    )(_gather_body)
    out2d = f(indices_tmajor, weight)
    return out2d.reshape(SEQ, BATCH, D).transpose(1, 0, 2)


def kernel(indices, weight):
    return _embed(indices.T, weight)


# speedup vs baseline: 10.8552x; 1.0019x over previous
import functools

import jax
import jax.numpy as jnp
from jax import lax
from jax.experimental import pallas as pl
from jax.experimental.pallas import tpu as pltpu
from jax.experimental.pallas import tpu_sc as plsc

VOCAB = 100000
D = 128
BATCH = 4096
SEQ = 50
B_TOTAL = BATCH * SEQ

_info = plsc.get_sparse_core_info()
NC = _info.num_cores
NS = _info.num_subcores
NW = NC * NS
B_PER_W = B_TOTAL // NW
CHUNK = 128
N_CHUNKS = B_PER_W // CHUNK
NBUF = 7
K = 2
N_ROUNDS = -(-N_CHUNKS // NBUF)


def _gather_body(idx_hbm, table_hbm, out_hbm, idx_v, rows_v, gsems, wsems):
    wid = lax.axis_index("s") * NC + lax.axis_index("c")

    pltpu.sync_copy(idx_hbm.at[:, pl.ds(wid * CHUNK, CHUNK)], idx_v)

    for b in range(NBUF):
        pltpu.async_copy(table_hbm.at[idx_v.at[b]], rows_v.at[b], gsems.at[b])

    def wdesc(g, b):
        return pltpu.make_async_copy(
            rows_v.at[b],
            out_hbm.at[pl.ds(g * BATCH + wid * CHUNK, CHUNK)],
            wsems.at[b])

    def round_(rnd, carry):
        for b in range(NBUF):
            g = rnd * NBUF + b

            @pl.when(g < N_CHUNKS)
            def _():
                pltpu.make_async_copy(table_hbm.at[idx_v.at[g]], rows_v.at[b],
                                      gsems.at[b]).wait()
                pltpu.async_copy(
                    rows_v.at[b],
                    out_hbm.at[pl.ds(g * BATCH + wid * CHUNK, CHUNK)],
                    wsems.at[b])

            w = g - K
            bw = (b - K) % NBUF

            @pl.when((w >= 0) & (w + NBUF < N_CHUNKS))
            def _():
                wdesc(w, bw).wait()
                pltpu.async_copy(table_hbm.at[idx_v.at[w + NBUF]],
                                 rows_v.at[bw], gsems.at[bw])
        return carry

    lax.fori_loop(0, N_ROUNDS, round_, 0)

    for g in range(N_CHUNKS - NBUF, N_CHUNKS):
        wdesc(g, g % NBUF).wait()


@jax.jit
def _embed(indices_tmajor, weight):
    mesh = plsc.VectorSubcoreMesh(core_axis_name="c", subcore_axis_name="s")
    f = functools.partial(
        pl.kernel,
        out_type=jax.ShapeDtypeStruct((B_TOTAL, D), jnp.float32),
        mesh=mesh,
        scratch_types=[
            pltpu.VMEM((SEQ, CHUNK), jnp.int32),
            pltpu.VMEM((NBUF, CHUNK, D), jnp.float32),
            pltpu.SemaphoreType.DMA((NBUF,)),
            pltpu.SemaphoreType.DMA((NBUF,)),
        ],
    )(_gather_body)
    out2d = f(indices_tmajor, weight)
    return out2d.reshape(SEQ, BATCH, D).transpose(1, 0, 2)


def kernel(indices, weight):
    return _embed(indices.T, weight)
